# Initial kernel scaffold; baseline (speedup 1.0000x reference)
import jax
import jax.numpy as jnp
from jax.experimental import pallas as pl


def kernel(user_id, history, top_genres, item_id, tmdb_genres, user_avg_rating, user_activity, release_year, item_avg_rating, revenue, item_table, genre_table, user_table, user_cont_W, user_cont_b, user_W1, user_b1, user_W2, user_b2, item_cont_W, item_cont_b, item_W1, item_b1, item_W2, item_b2):
    raise NotImplementedError("write your pallas kernel here")


# speedup vs baseline: 1.9685x; 1.9685x over previous
import functools

import numpy as np
import jax
import jax.numpy as jnp
from jax import lax
from jax.experimental import pallas as pl
from jax.experimental.pallas import tpu as pltpu
from jax.experimental.pallas import tpu_sc as plsc

B = 16384
H = 50
G = 8
D = 64

NC = 2
NS = 16
NW = NC * NS
RPW = B // NW
IPW = RPW * H
CHUNK = 128
NCH = IPW // CHUNK
NSM = RPW // CHUNK

_TGT = ((np.arange(NW, dtype=np.int32) // NC)[:, None] * RPW
        + (np.arange(IPW, dtype=np.int32) // H)[None, :]).reshape(NW, NCH, CHUNK)


def _sc_gather(item_table, user_table, hist_idx, tgt_idx, uidx, iidx):
    mesh = plsc.VectorSubcoreMesh(core_axis_name="c", subcore_axis_name="s")
    f32 = jnp.float32

    @functools.partial(
        pl.kernel,
        out_type=(
            jax.ShapeDtypeStruct((B, D), f32),
            jax.ShapeDtypeStruct((B, D), f32),
            jax.ShapeDtypeStruct((B, D), f32),
        ),
        mesh=mesh,
        compiler_params=pltpu.CompilerParams(use_tc_tiling_on_sc=False),
        scratch_types=[
            pltpu.VMEM((NCH, CHUNK), jnp.int32),
            pltpu.VMEM((NCH, CHUNK), jnp.int32),
            pltpu.VMEM((CHUNK, D), f32),
            pltpu.VMEM((NSM, CHUNK), jnp.int32),
            pltpu.VMEM_SHARED((NS * RPW, D), f32),
        ],
    )
    def k(item_hbm, user_hbm, hist_hbm, tgt_hbm, uidx_hbm, iidx_hbm,
          hist_out, u_out, i_out, idx_v, tgt_v, buf, sidx_v, acc):
        c = lax.axis_index("c")
        s = lax.axis_index("s")
        wid = s * NC + c
        base = wid * RPW

        zero16 = jnp.zeros((16,), f32)

        def zrow(r, carry):
            for q in range(D // 16):
                buf[r, pl.ds(q * 16, 16)] = zero16
            return carry

        lax.fori_loop(0, CHUNK, zrow, 0)
        for j in range(NSM):
            pltpu.sync_copy(buf, acc.at[pl.ds(s * RPW + j * CHUNK, CHUNK)])

        pltpu.sync_copy(hist_hbm.at[wid], idx_v)
        pltpu.sync_copy(tgt_hbm.at[wid], tgt_v)

        def body(ci, carry):
            pltpu.sync_copy(item_hbm.at[idx_v.at[ci]], buf)
            pltpu.sync_copy(buf, acc.at[tgt_v.at[ci]], add=True)
            return carry

        lax.fori_loop(0, NCH, body, 0)
        pltpu.sync_copy(acc.at[pl.ds(s * RPW, RPW)],
                        hist_out.at[pl.ds(base, RPW)])

        pltpu.sync_copy(uidx_hbm.at[wid], sidx_v)
        for j in range(NSM):
            pltpu.sync_copy(user_hbm.at[sidx_v.at[j]], buf)
            pltpu.sync_copy(buf, u_out.at[pl.ds(base + j * CHUNK, CHUNK)])
        pltpu.sync_copy(iidx_hbm.at[wid], sidx_v)
        for j in range(NSM):
            pltpu.sync_copy(item_hbm.at[sidx_v.at[j]], buf)
            pltpu.sync_copy(buf, i_out.at[pl.ds(base + j * CHUNK, CHUNK)])

    return k(item_table, user_table, hist_idx, tgt_idx, uidx, iidx)


MB = 1024


def _tower_kernel(hist_ref, u_ref, i_ref, hids_ref, tg_ref, tmg_ref,
                  uar_ref, uac_ref, ry_ref, iar_ref, rev_ref, gt_ref,
                  ucw_ref, ucb_ref, uw1a_ref, uw1b_ref, uw1c_ref, uw1d_ref,
                  ub1_ref, uw2_ref, ub2_ref,
                  icw_ref, icb_ref, iw1a_ref, iw1b_ref, iw1c_ref,
                  ib1_ref, iw2_ref, ib2_ref,
                  uo_ref, io_ref):
    f32 = jnp.float32

    def onehot(ids):
        iota = lax.broadcasted_iota(jnp.int32, (MB, 32), 1)
        oh = jnp.zeros((MB, 32), f32)
        for g in range(G):
            oh = oh + (ids[:, g:g + 1] == iota).astype(f32)
        return oh

    def l2norm(x):
        n = jnp.sqrt(jnp.sum(x * x, axis=1, keepdims=True))
        return x / jnp.maximum(n, 1e-12)

    gt = gt_ref[...]

    hcnt = jnp.sum((hids_ref[...] > 0).astype(f32), axis=1, keepdims=True)
    hist_emb = hist_ref[...] / (hcnt + 1e-8)
    tg = tg_ref[...]
    gcnt = jnp.sum((tg > 0).astype(f32), axis=1, keepdims=True)
    ug_emb = jnp.dot(onehot(tg), gt, preferred_element_type=f32, precision=lax.Precision.HIGHEST) / (gcnt + 1e-8)
    ucw = ucw_ref[...]
    u_cont = jnp.maximum(uar_ref[...] * ucw[0:1, :] + uac_ref[...] * ucw[1:2, :]
                         + ucb_ref[...], 0.0)
    u_h = jnp.maximum(
        jnp.dot(u_ref[...], uw1a_ref[...], preferred_element_type=f32, precision=lax.Precision.HIGHEST)
        + jnp.dot(hist_emb, uw1b_ref[...], preferred_element_type=f32, precision=lax.Precision.HIGHEST)
        + jnp.dot(ug_emb, uw1c_ref[...], preferred_element_type=f32, precision=lax.Precision.HIGHEST)
        + jnp.dot(u_cont, uw1d_ref[...], preferred_element_type=f32, precision=lax.Precision.HIGHEST)
        + ub1_ref[...], 0.0)
    uo_ref[...] = l2norm(jnp.dot(u_h, uw2_ref[...], preferred_element_type=f32, precision=lax.Precision.HIGHEST)
                         + ub2_ref[...])

    tmg = tmg_ref[...]
    igcnt = jnp.sum((tmg > 0).astype(f32), axis=1, keepdims=True)
    ig_emb = jnp.dot(onehot(tmg), gt, preferred_element_type=f32, precision=lax.Precision.HIGHEST) / (igcnt + 1e-8)
    icw = icw_ref[...]
    i_cont = jnp.maximum(ry_ref[...] * icw[0:1, :] + iar_ref[...] * icw[1:2, :]
                         + rev_ref[...] * icw[2:3, :] + icb_ref[...], 0.0)
    i_h = jnp.maximum(
        jnp.dot(i_ref[...], iw1a_ref[...], preferred_element_type=f32, precision=lax.Precision.HIGHEST)
        + jnp.dot(ig_emb, iw1b_ref[...], preferred_element_type=f32, precision=lax.Precision.HIGHEST)
        + jnp.dot(i_cont, iw1c_ref[...], preferred_element_type=f32, precision=lax.Precision.HIGHEST)
        + ib1_ref[...], 0.0)
    io_ref[...] = l2norm(jnp.dot(i_h, iw2_ref[...], preferred_element_type=f32, precision=lax.Precision.HIGHEST)
                         + ib2_ref[...])


def kernel(user_id, history, top_genres, item_id, tmdb_genres,
           user_avg_rating, user_activity, release_year, item_avg_rating,
           revenue, item_table, genre_table, user_table,
           user_cont_W, user_cont_b, user_W1, user_b1, user_W2, user_b2,
           item_cont_W, item_cont_b, item_W1, item_b1, item_W2, item_b2):
    f32 = jnp.float32

    hist_idx = history.reshape(NW, NCH, CHUNK)
    tgt_idx = jnp.asarray(_TGT)
    uidx = user_id.reshape(NW, NSM, CHUNK)
    iidx = item_id.reshape(NW, NSM, CHUNK)

    hist_sum, u_rows, i_rows = _sc_gather(
        item_table, user_table, hist_idx, tgt_idx, uidx, iidx)

    gt_pad = jnp.zeros((32, D), f32).at[:21].set(genre_table)
    col = lambda x: x.reshape(B, 1).astype(f32)
    row = lambda x: x.reshape(1, -1)

    grid = (B // MB,)
    bspec = lambda cols: pl.BlockSpec((MB, cols), lambda i: (i, 0))
    wspec = lambda shape: pl.BlockSpec(shape, lambda i: (0, 0))

    uw1 = [user_W1[j * D:(j + 1) * D] for j in range(4)]
    iw1 = [item_W1[j * D:(j + 1) * D] for j in range(3)]

    uo, io = pl.pallas_call(
        _tower_kernel,
        grid=grid,
        in_specs=[
            bspec(D), bspec(D), bspec(D), bspec(H), bspec(G), bspec(G),
            bspec(1), bspec(1), bspec(1), bspec(1), bspec(1),
            wspec((32, D)),
            wspec((2, D)), wspec((1, D)),
            wspec((D, 128)), wspec((D, 128)), wspec((D, 128)), wspec((D, 128)),
            wspec((1, 128)), wspec((128, D)), wspec((1, D)),
            wspec((3, D)), wspec((1, D)),
            wspec((D, 128)), wspec((D, 128)), wspec((D, 128)),
            wspec((1, 128)), wspec((128, D)), wspec((1, D)),
        ],
        out_specs=[bspec(D), bspec(D)],
        out_shape=[
            jax.ShapeDtypeStruct((B, D), f32),
            jax.ShapeDtypeStruct((B, D), f32),
        ],
    )(hist_sum, u_rows, i_rows, history, top_genres, tmdb_genres,
      col(user_avg_rating), col(user_activity), col(release_year),
      col(item_avg_rating), col(revenue), gt_pad,
      user_cont_W, row(user_cont_b), uw1[0], uw1[1], uw1[2], uw1[3],
      row(user_b1), user_W2, row(user_b2),
      item_cont_W, row(item_cont_b), iw1[0], iw1[1], iw1[2],
      row(item_b1), item_W2, row(item_b2))
    return (uo, io)


# speedup vs baseline: 2.1664x; 1.1006x over previous
import functools

import numpy as np
import jax
import jax.numpy as jnp
from jax import lax
from jax.experimental import pallas as pl
from jax.experimental.pallas import tpu as pltpu
from jax.experimental.pallas import tpu_sc as plsc

B = 16384
H = 50
G = 8
D = 64

NC = 2
NS = 16
NW = NC * NS
RPW = B // NW
IPW = RPW * H
CHUNK = 128
NCH = IPW // CHUNK
NSM = RPW // CHUNK

_TGT = ((np.arange(NW, dtype=np.int32) // NC)[:, None] * RPW
        + (np.arange(IPW, dtype=np.int32) // H)[None, :]).reshape(NW, NCH, CHUNK)


NBUF = 4
NOUT = NCH // NBUF


def _sc_gather(item_table, user_table, hist_idx, tgt_idx, uidx, iidx):
    mesh = plsc.VectorSubcoreMesh(core_axis_name="c", subcore_axis_name="s")
    f32 = jnp.float32

    @functools.partial(
        pl.kernel,
        out_type=(
            jax.ShapeDtypeStruct((B, D), f32),
            jax.ShapeDtypeStruct((B, D), f32),
            jax.ShapeDtypeStruct((B, D), f32),
        ),
        mesh=mesh,
        compiler_params=pltpu.CompilerParams(use_tc_tiling_on_sc=False),
        scratch_types=(
            [pltpu.VMEM((NCH, CHUNK), jnp.int32),
             pltpu.VMEM((NCH, CHUNK), jnp.int32),
             pltpu.VMEM((NSM, CHUNK), jnp.int32),
             pltpu.VMEM_SHARED((NS * RPW, D), f32)]
            + [pltpu.VMEM((CHUNK, D), f32) for _ in range(NBUF)]
            + [pltpu.SemaphoreType.DMA for _ in range(2 * NBUF + 1)]
        ),
    )
    def k(item_hbm, user_hbm, hist_hbm, tgt_hbm, uidx_hbm, iidx_hbm,
          hist_out, u_out, i_out, idx_v, tgt_v, sidx_v, acc, *bufsem):
        bufs = bufsem[:NBUF]
        gsem = bufsem[NBUF:2 * NBUF]
        ssem = bufsem[2 * NBUF:3 * NBUF]
        osem = bufsem[3 * NBUF]
        c = lax.axis_index("c")
        s = lax.axis_index("s")
        wid = s * NC + c
        base = wid * RPW

        zero16 = jnp.zeros((16,), f32)

        def zrow(r, carry):
            for q in range(D // 16):
                bufs[0][r, pl.ds(q * 16, 16)] = zero16
            return carry

        lax.fori_loop(0, CHUNK, zrow, 0)
        for j in range(NSM):
            pltpu.sync_copy(bufs[0], acc.at[pl.ds(s * RPW + j * CHUNK, CHUNK)])

        pltpu.sync_copy(hist_hbm.at[wid], idx_v)
        pltpu.sync_copy(tgt_hbm.at[wid], tgt_v)

        def gwait(b, idx=None):
            src = item_hbm.at[idx_v.at[0] if idx is None else idx]
            pltpu.make_async_copy(src, bufs[b], gsem[b]).wait()

        def swait(b):
            pltpu.make_async_copy(bufs[b], acc.at[tgt_v.at[0]],
                                  ssem[b]).wait()

        for b in range(NBUF):
            pltpu.async_copy(item_hbm.at[idx_v.at[b]], bufs[b], gsem[b])

        def outer(g, carry):
            for b in range(NBUF):
                gwait(b)
                pltpu.async_copy(bufs[b], acc.at[tgt_v.at[g * NBUF + b]],
                                 ssem[b], add=True)
            for b in range(NBUF):
                swait(b)
                ci2 = lax.rem(g * NBUF + b + NBUF, NCH)
                pltpu.async_copy(item_hbm.at[idx_v.at[ci2]], bufs[b], gsem[b])
            return carry

        lax.fori_loop(0, NOUT, outer, 0)
        for b in range(NBUF):
            gwait(b)

        pltpu.async_copy(acc.at[pl.ds(s * RPW, RPW)],
                         hist_out.at[pl.ds(base, RPW)], osem)

        pltpu.sync_copy(uidx_hbm.at[wid], sidx_v)
        for j in range(NSM):
            pltpu.async_copy(user_hbm.at[sidx_v.at[j]], bufs[j], gsem[j])
        for j in range(NSM):
            gwait(j)
            pltpu.async_copy(bufs[j], u_out.at[pl.ds(base + j * CHUNK, CHUNK)],
                             ssem[j])
        pltpu.sync_copy(iidx_hbm.at[wid], sidx_v)
        for j in range(NSM):
            pltpu.make_async_copy(item_hbm.at[pl.ds(0, CHUNK)],
                                  u_out.at[pl.ds(0, CHUNK)], ssem[j]).wait()
            pltpu.async_copy(item_hbm.at[sidx_v.at[j]], bufs[j], gsem[j])
        for j in range(NSM):
            gwait(j)
            pltpu.async_copy(bufs[j], i_out.at[pl.ds(base + j * CHUNK, CHUNK)],
                             ssem[j])
        for j in range(NSM):
            pltpu.make_async_copy(item_hbm.at[pl.ds(0, CHUNK)],
                                  i_out.at[pl.ds(0, CHUNK)], ssem[j]).wait()
        pltpu.make_async_copy(acc.at[pl.ds(s * RPW, RPW)],
                              hist_out.at[pl.ds(base, RPW)], osem).wait()

    return k(item_table, user_table, hist_idx, tgt_idx, uidx, iidx)


MB = 1024


def _tower_kernel(hist_ref, u_ref, i_ref, hids_ref, tg_ref, tmg_ref,
                  uar_ref, uac_ref, ry_ref, iar_ref, rev_ref, gt_ref,
                  ucw_ref, ucb_ref, uw1a_ref, uw1b_ref, uw1c_ref, uw1d_ref,
                  ub1_ref, uw2_ref, ub2_ref,
                  icw_ref, icb_ref, iw1a_ref, iw1b_ref, iw1c_ref,
                  ib1_ref, iw2_ref, ib2_ref,
                  uo_ref, io_ref):
    f32 = jnp.float32

    def onehot(ids):
        iota = lax.broadcasted_iota(jnp.int32, (MB, 32), 1)
        oh = jnp.zeros((MB, 32), f32)
        for g in range(G):
            oh = oh + (ids[:, g:g + 1] == iota).astype(f32)
        return oh

    def l2norm(x):
        n = jnp.sqrt(jnp.sum(x * x, axis=1, keepdims=True))
        return x / jnp.maximum(n, 1e-12)

    gt = gt_ref[...]

    hcnt = jnp.sum((hids_ref[...] > 0).astype(f32), axis=1, keepdims=True)
    hist_emb = hist_ref[...] / (hcnt + 1e-8)
    tg = tg_ref[...]
    gcnt = jnp.sum((tg > 0).astype(f32), axis=1, keepdims=True)
    ug_emb = jnp.dot(onehot(tg), gt, preferred_element_type=f32, precision=lax.Precision.HIGHEST) / (gcnt + 1e-8)
    ucw = ucw_ref[...]
    u_cont = jnp.maximum(uar_ref[...] * ucw[0:1, :] + uac_ref[...] * ucw[1:2, :]
                         + ucb_ref[...], 0.0)
    u_h = jnp.maximum(
        jnp.dot(u_ref[...], uw1a_ref[...], preferred_element_type=f32, precision=lax.Precision.HIGHEST)
        + jnp.dot(hist_emb, uw1b_ref[...], preferred_element_type=f32, precision=lax.Precision.HIGHEST)
        + jnp.dot(ug_emb, uw1c_ref[...], preferred_element_type=f32, precision=lax.Precision.HIGHEST)
        + jnp.dot(u_cont, uw1d_ref[...], preferred_element_type=f32, precision=lax.Precision.HIGHEST)
        + ub1_ref[...], 0.0)
    uo_ref[...] = l2norm(jnp.dot(u_h, uw2_ref[...], preferred_element_type=f32, precision=lax.Precision.HIGHEST)
                         + ub2_ref[...])

    tmg = tmg_ref[...]
    igcnt = jnp.sum((tmg > 0).astype(f32), axis=1, keepdims=True)
    ig_emb = jnp.dot(onehot(tmg), gt, preferred_element_type=f32, precision=lax.Precision.HIGHEST) / (igcnt + 1e-8)
    icw = icw_ref[...]
    i_cont = jnp.maximum(ry_ref[...] * icw[0:1, :] + iar_ref[...] * icw[1:2, :]
                         + rev_ref[...] * icw[2:3, :] + icb_ref[...], 0.0)
    i_h = jnp.maximum(
        jnp.dot(i_ref[...], iw1a_ref[...], preferred_element_type=f32, precision=lax.Precision.HIGHEST)
        + jnp.dot(ig_emb, iw1b_ref[...], preferred_element_type=f32, precision=lax.Precision.HIGHEST)
        + jnp.dot(i_cont, iw1c_ref[...], preferred_element_type=f32, precision=lax.Precision.HIGHEST)
        + ib1_ref[...], 0.0)
    io_ref[...] = l2norm(jnp.dot(i_h, iw2_ref[...], preferred_element_type=f32, precision=lax.Precision.HIGHEST)
                         + ib2_ref[...])


def kernel(user_id, history, top_genres, item_id, tmdb_genres,
           user_avg_rating, user_activity, release_year, item_avg_rating,
           revenue, item_table, genre_table, user_table,
           user_cont_W, user_cont_b, user_W1, user_b1, user_W2, user_b2,
           item_cont_W, item_cont_b, item_W1, item_b1, item_W2, item_b2):
    f32 = jnp.float32

    hist_idx = history.reshape(NW, NCH, CHUNK)
    tgt_idx = jnp.asarray(_TGT)
    uidx = user_id.reshape(NW, NSM, CHUNK)
    iidx = item_id.reshape(NW, NSM, CHUNK)

    hist_sum, u_rows, i_rows = _sc_gather(
        item_table, user_table, hist_idx, tgt_idx, uidx, iidx)

    gt_pad = jnp.zeros((32, D), f32).at[:21].set(genre_table)
    col = lambda x: x.reshape(B, 1).astype(f32)
    row = lambda x: x.reshape(1, -1)

    grid = (B // MB,)
    bspec = lambda cols: pl.BlockSpec((MB, cols), lambda i: (i, 0))
    wspec = lambda shape: pl.BlockSpec(shape, lambda i: (0, 0))

    uw1 = [user_W1[j * D:(j + 1) * D] for j in range(4)]
    iw1 = [item_W1[j * D:(j + 1) * D] for j in range(3)]

    uo, io = pl.pallas_call(
        _tower_kernel,
        grid=grid,
        in_specs=[
            bspec(D), bspec(D), bspec(D), bspec(H), bspec(G), bspec(G),
            bspec(1), bspec(1), bspec(1), bspec(1), bspec(1),
            wspec((32, D)),
            wspec((2, D)), wspec((1, D)),
            wspec((D, 128)), wspec((D, 128)), wspec((D, 128)), wspec((D, 128)),
            wspec((1, 128)), wspec((128, D)), wspec((1, D)),
            wspec((3, D)), wspec((1, D)),
            wspec((D, 128)), wspec((D, 128)), wspec((D, 128)),
            wspec((1, 128)), wspec((128, D)), wspec((1, D)),
        ],
        out_specs=[bspec(D), bspec(D)],
        out_shape=[
            jax.ShapeDtypeStruct((B, D), f32),
            jax.ShapeDtypeStruct((B, D), f32),
        ],
    )(hist_sum, u_rows, i_rows, history, top_genres, tmdb_genres,
      col(user_avg_rating), col(user_activity), col(release_year),
      col(item_avg_rating), col(revenue), gt_pad,
      user_cont_W, row(user_cont_b), uw1[0], uw1[1], uw1[2], uw1[3],
      row(user_b1), user_W2, row(user_b2),
      item_cont_W, row(item_cont_b), iw1[0], iw1[1], iw1[2],
      row(item_b1), item_W2, row(item_b2))
    return (uo, io)


# speedup vs baseline: 2.5429x; 1.1738x over previous
import functools

import numpy as np
import jax
import jax.numpy as jnp
from jax import lax
from jax.experimental import pallas as pl
from jax.experimental.pallas import tpu as pltpu
from jax.experimental.pallas import tpu_sc as plsc

B = 16384
H = 50
G = 8
D = 64
V = 1000001

NC = 2
NS = 16
NW = NC * NS
RPW = B // NW
IPW = RPW * H
CHUNK = 128
NCH = IPW // CHUNK
NSM = RPW // CHUNK
NBUF = 4
P = 2
CPP = NCH // P
RPP = RPW // P
NOUTP = CPP // NBUF

CPACK = 2048
VP = ((V + CPACK - 1) // CPACK) * CPACK

_TGT = ((np.arange(NW, dtype=np.int32) // NC)[:, None] * RPP
        + (np.arange(CPP * CHUNK, dtype=np.int32) // H)[None, :]
        ).reshape(NW, CPP, CHUNK)


def _pack_body(tin_ref, out_ref):
    x = tin_ref[...]
    out_ref[...] = jnp.concatenate(
        [x.T, jnp.zeros((CPACK, D), jnp.float32)], axis=1)


def _pack(table_t):
    return pl.pallas_call(
        _pack_body,
        grid=(VP // CPACK,),
        in_specs=[pl.BlockSpec((D, CPACK), lambda j: (0, j))],
        out_specs=pl.BlockSpec((CPACK, 2 * D), lambda j: (j, 0)),
        out_shape=jax.ShapeDtypeStruct((VP, 2 * D), jnp.float32),
    )(table_t)


_MESH = plsc.VectorSubcoreMesh(core_axis_name="c", subcore_axis_name="s")
_F32 = jnp.float32


def _sc_hist(item_packed, hist_idx, tgt_idx, iidx):
    @functools.partial(
        pl.kernel,
        out_type=(
            jax.ShapeDtypeStruct((B, 2 * D), _F32),
            jax.ShapeDtypeStruct((B, 2 * D), _F32),
        ),
        mesh=_MESH,
        scratch_types=(
            [pltpu.VMEM((CPP, CHUNK), jnp.int32),
             pltpu.VMEM((CPP, CHUNK), jnp.int32),
             pltpu.VMEM((NSM, CHUNK), jnp.int32),
             pltpu.VMEM_SHARED((NS * RPP, 2 * D), _F32)]
            + [pltpu.VMEM((CHUNK, 2 * D), _F32) for _ in range(NBUF)]
            + [pltpu.SemaphoreType.DMA for _ in range(2 * NBUF + 1)]
        ),
    )
    def k(item_hbm, hist_hbm, tgt_hbm, iidx_hbm,
          hist_out, i_out, idx_v, tgt_v, sidx_v, acc, *bufsem):
        bufs = bufsem[:NBUF]
        gsem = bufsem[NBUF:2 * NBUF]
        ssem = bufsem[2 * NBUF:3 * NBUF]
        osem = bufsem[3 * NBUF]
        c = lax.axis_index("c")
        s = lax.axis_index("s")
        wid = s * NC + c
        base = wid * RPW
        zero16 = jnp.zeros((16,), _F32)

        def gwait(b):
            pltpu.make_async_copy(item_hbm.at[idx_v.at[0]], bufs[b],
                                  gsem[b]).wait()

        def swait(b):
            pltpu.make_async_copy(bufs[b], acc.at[tgt_v.at[0]],
                                  ssem[b]).wait()

        pltpu.sync_copy(tgt_hbm.at[wid], tgt_v)

        for p in range(P):
            if p > 0:
                pltpu.make_async_copy(acc.at[pl.ds(s * RPP, RPP)],
                                      hist_out.at[pl.ds(base, RPP)],
                                      osem).wait()
            def zrow(r, carry):
                for q in range(2 * D // 16):
                    bufs[0][r, pl.ds(q * 16, 16)] = zero16
                return carry

            lax.fori_loop(0, CHUNK, zrow, 0)
            for j in range(RPP // CHUNK):
                pltpu.sync_copy(bufs[0],
                                acc.at[pl.ds(s * RPP + j * CHUNK, CHUNK)])
            pltpu.sync_copy(hist_hbm.at[wid, p], idx_v)

            for b in range(NBUF):
                pltpu.async_copy(item_hbm.at[idx_v.at[b]], bufs[b], gsem[b])

            def outer(g, carry):
                for b in range(NBUF):
                    gwait(b)
                    pltpu.async_copy(bufs[b], acc.at[tgt_v.at[g * NBUF + b]],
                                     ssem[b], add=True)
                for b in range(NBUF):
                    swait(b)
                    ci2 = lax.rem(g * NBUF + b + NBUF, CPP)
                    pltpu.async_copy(item_hbm.at[idx_v.at[ci2]], bufs[b],
                                     gsem[b])
                return carry

            lax.fori_loop(0, NOUTP, outer, 0)
            for b in range(NBUF):
                gwait(b)

            pltpu.async_copy(acc.at[pl.ds(s * RPP, RPP)],
                             hist_out.at[pl.ds(base + p * RPP, RPP)], osem)

        pltpu.sync_copy(iidx_hbm.at[wid], sidx_v)
        for j in range(NSM):
            pltpu.async_copy(item_hbm.at[sidx_v.at[j]], bufs[j], gsem[j])
        for j in range(NSM):
            gwait(j)
            pltpu.async_copy(bufs[j], i_out.at[pl.ds(base + j * CHUNK, CHUNK)],
                             ssem[j])
        for j in range(NSM):
            pltpu.make_async_copy(item_hbm.at[pl.ds(0, CHUNK)],
                                  i_out.at[pl.ds(0, CHUNK)], ssem[j]).wait()
        pltpu.make_async_copy(acc.at[pl.ds(s * RPP, RPP)],
                              hist_out.at[pl.ds(base, RPP)], osem).wait()

    return k(item_packed, hist_idx, tgt_idx, iidx)


def _sc_user(user_packed, uidx):
    @functools.partial(
        pl.kernel,
        out_type=jax.ShapeDtypeStruct((B, 2 * D), _F32),
        mesh=_MESH,
        scratch_types=(
            [pltpu.VMEM((NSM, CHUNK), jnp.int32)]
            + [pltpu.VMEM((CHUNK, 2 * D), _F32) for _ in range(NSM)]
            + [pltpu.SemaphoreType.DMA for _ in range(2 * NSM)]
        ),
    )
    def k(user_hbm, uidx_hbm, u_out, sidx_v, *bufsem):
        bufs = bufsem[:NSM]
        gsem = bufsem[NSM:2 * NSM]
        ssem = bufsem[2 * NSM:3 * NSM]
        c = lax.axis_index("c")
        s = lax.axis_index("s")
        wid = s * NC + c
        base = wid * RPW
        pltpu.sync_copy(uidx_hbm.at[wid], sidx_v)
        for j in range(NSM):
            pltpu.async_copy(user_hbm.at[sidx_v.at[j]], bufs[j], gsem[j])
        for j in range(NSM):
            pltpu.make_async_copy(user_hbm.at[sidx_v.at[0]], bufs[j],
                                  gsem[j]).wait()
            pltpu.async_copy(bufs[j], u_out.at[pl.ds(base + j * CHUNK, CHUNK)],
                             ssem[j])
        for j in range(NSM):
            pltpu.make_async_copy(user_hbm.at[pl.ds(0, CHUNK)],
                                  u_out.at[pl.ds(0, CHUNK)], ssem[j]).wait()

    return k(user_packed, uidx)


MB = 1024


def _tower_kernel(hist_ref, u_ref, i_ref, hids_ref, tg_ref, tmg_ref,
                  uar_ref, uac_ref, ry_ref, iar_ref, rev_ref, gt_ref,
                  ucw_ref, ucb_ref, uw1a_ref, uw1b_ref, uw1c_ref, uw1d_ref,
                  ub1_ref, uw2_ref, ub2_ref,
                  icw_ref, icb_ref, iw1a_ref, iw1b_ref, iw1c_ref,
                  ib1_ref, iw2_ref, ib2_ref,
                  uo_ref, io_ref):
    f32 = jnp.float32
    hi = lax.Precision.HIGHEST

    def onehot(ids):
        iota = lax.broadcasted_iota(jnp.int32, (MB, 32), 1)
        oh = jnp.zeros((MB, 32), f32)
        for g in range(G):
            oh = oh + (ids[:, g:g + 1] == iota).astype(f32)
        return oh

    def l2norm(x):
        n = jnp.sqrt(jnp.sum(x * x, axis=1, keepdims=True))
        return x / jnp.maximum(n, 1e-12)

    gt = gt_ref[...]

    hcnt = jnp.sum((hids_ref[...] > 0).astype(f32), axis=1, keepdims=True)
    hist_emb = hist_ref[:, :D] / (hcnt + 1e-8)
    tg = tg_ref[...]
    gcnt = jnp.sum((tg > 0).astype(f32), axis=1, keepdims=True)
    ug_emb = jnp.dot(onehot(tg), gt, preferred_element_type=f32,
                     precision=hi) / (gcnt + 1e-8)
    ucw = ucw_ref[...]
    u_cont = jnp.maximum(uar_ref[...] * ucw[0:1, :] + uac_ref[...] * ucw[1:2, :]
                         + ucb_ref[...], 0.0)
    u_h = jnp.maximum(
        jnp.dot(u_ref[:, :D], uw1a_ref[...], preferred_element_type=f32,
                precision=hi)
        + jnp.dot(hist_emb, uw1b_ref[...], preferred_element_type=f32,
                  precision=hi)
        + jnp.dot(ug_emb, uw1c_ref[...], preferred_element_type=f32,
                  precision=hi)
        + jnp.dot(u_cont, uw1d_ref[...], preferred_element_type=f32,
                  precision=hi)
        + ub1_ref[...], 0.0)
    uo_ref[...] = l2norm(jnp.dot(u_h, uw2_ref[...], preferred_element_type=f32,
                                 precision=hi) + ub2_ref[...])

    tmg = tmg_ref[...]
    igcnt = jnp.sum((tmg > 0).astype(f32), axis=1, keepdims=True)
    ig_emb = jnp.dot(onehot(tmg), gt, preferred_element_type=f32,
                     precision=hi) / (igcnt + 1e-8)
    icw = icw_ref[...]
    i_cont = jnp.maximum(ry_ref[...] * icw[0:1, :] + iar_ref[...] * icw[1:2, :]
                         + rev_ref[...] * icw[2:3, :] + icb_ref[...], 0.0)
    i_h = jnp.maximum(
        jnp.dot(i_ref[:, :D], iw1a_ref[...], preferred_element_type=f32,
                precision=hi)
        + jnp.dot(ig_emb, iw1b_ref[...], preferred_element_type=f32,
                  precision=hi)
        + jnp.dot(i_cont, iw1c_ref[...], preferred_element_type=f32,
                  precision=hi)
        + ib1_ref[...], 0.0)
    io_ref[...] = l2norm(jnp.dot(i_h, iw2_ref[...], preferred_element_type=f32,
                                 precision=hi) + ib2_ref[...])


def kernel(user_id, history, top_genres, item_id, tmdb_genres,
           user_avg_rating, user_activity, release_year, item_avg_rating,
           revenue, item_table, genre_table, user_table,
           user_cont_W, user_cont_b, user_W1, user_b1, user_W2, user_b2,
           item_cont_W, item_cont_b, item_W1, item_b1, item_W2, item_b2):
    f32 = jnp.float32

    hist_idx = history.reshape(NW, P, CPP, CHUNK)
    tgt_idx = jnp.asarray(_TGT)
    uidx = user_id.reshape(NW, NSM, CHUNK)
    iidx = item_id.reshape(NW, NSM, CHUNK)

    item_packed = _pack(item_table.T)
    hist_sum, i_rows = _sc_hist(item_packed, hist_idx, tgt_idx, iidx)
    user_packed = _pack(user_table.T)
    u_rows = _sc_user(user_packed, uidx)

    gt_pad = jnp.zeros((32, D), f32).at[:21].set(genre_table)
    col = lambda x: x.reshape(B, 1).astype(f32)
    row = lambda x: x.reshape(1, -1)

    grid = (B // MB,)
    bspec = lambda cols: pl.BlockSpec((MB, cols), lambda i: (i, 0))
    wspec = lambda shape: pl.BlockSpec(shape, lambda i: (0, 0))

    uw1 = [user_W1[j * D:(j + 1) * D] for j in range(4)]
    iw1 = [item_W1[j * D:(j + 1) * D] for j in range(3)]

    uo, io = pl.pallas_call(
        _tower_kernel,
        grid=grid,
        in_specs=[
            bspec(2 * D), bspec(2 * D), bspec(2 * D),
            bspec(H), bspec(G), bspec(G),
            bspec(1), bspec(1), bspec(1), bspec(1), bspec(1),
            wspec((32, D)),
            wspec((2, D)), wspec((1, D)),
            wspec((D, 128)), wspec((D, 128)), wspec((D, 128)), wspec((D, 128)),
            wspec((1, 128)), wspec((128, D)), wspec((1, D)),
            wspec((3, D)), wspec((1, D)),
            wspec((D, 128)), wspec((D, 128)), wspec((D, 128)),
            wspec((1, 128)), wspec((128, D)), wspec((1, D)),
        ],
        out_specs=[bspec(D), bspec(D)],
        out_shape=[
            jax.ShapeDtypeStruct((B, D), f32),
            jax.ShapeDtypeStruct((B, D), f32),
        ],
    )(hist_sum, u_rows, i_rows, history, top_genres, tmdb_genres,
      col(user_avg_rating), col(user_activity), col(release_year),
      col(item_avg_rating), col(revenue), gt_pad,
      user_cont_W, row(user_cont_b), uw1[0], uw1[1], uw1[2], uw1[3],
      row(user_b1), user_W2, row(user_b2),
      item_cont_W, row(item_cont_b), iw1[0], iw1[1], iw1[2],
      row(item_b1), item_W2, row(item_b2))
    return (uo, io)
